# baseline (speedup 1.0000x reference)
import functools

import jax
import jax.numpy as jnp
from jax import lax
from jax.experimental import pallas as pl
from jax.experimental.pallas import tpu as pltpu
from jax.experimental.pallas import tpu_sc as plsc

BV = 256
BO = 512
B_SC = 4
CV = 64


def _sc_pool_body(nv_hbm, x_hbm, o_hbm, nv_v, buf, stage, sems,
                  *, cv, fb, b0, nb, V):
    c = lax.axis_index("c")
    s = lax.axis_index("s")
    wid = s * 2 + c
    f0 = pl.multiple_of(wid * fb, 128)

    pltpu.sync_copy(nv_hbm, nv_v.at[pl.ds(0, nv_hbm.shape[0])])

    def do_sample(j, carry):
        b = b0 + j
        nv = jnp.minimum(nv_v[pl.ds(b, 16)][0], V)
        nchunks = (nv + cv - 1) // cv
        last0 = jnp.maximum(0, ((nv - cv + 7) // 8) * 8)

        def row0(i):
            return pl.multiple_of(jnp.minimum(i * cv, last0), 8)

        def start(i, slot):
            pltpu.make_async_copy(
                x_hbm.at[b, pl.ds(row0(i), cv), pl.ds(f0, fb)],
                buf.at[slot], sems.at[slot]).start()

        start(0, 0)

        def chunk(i, accs):
            slot = lax.rem(i, 2)

            @pl.when(i + 1 < nchunks)
            def _prefetch():
                start(i + 1, 1 - slot)

            pltpu.make_async_copy(
                x_hbm.at[b, pl.ds(0, cv), pl.ds(f0, fb)],
                buf.at[slot], sems.at[slot]).wait()
            nrows = jnp.minimum(cv, nv - row0(i))

            def rowstep(r, accs):
                return tuple(
                    jnp.maximum(a, buf[slot, r, pl.ds(f * 16, 16)])
                    for f, a in enumerate(accs)
                )

            return lax.fori_loop(0, nrows, rowstep, accs)

        neg = jnp.full((16,), -jnp.inf, jnp.float32)
        accs = tuple(neg for _ in range(fb // 16))
        accs = lax.fori_loop(0, nchunks, chunk, accs)
        for f, a in enumerate(accs):
            stage[pl.ds(f * 16, 16)] = a
        pltpu.sync_copy(stage, o_hbm.at[j, pl.ds(f0, fb)])
        return carry

    lax.fori_loop(0, nb, do_sample, 0)


def _pool_body(nv_ref, x_ref, o_ref, *, bv, max_views):
    b = pl.program_id(0)
    j = pl.program_id(1)
    nv = jnp.minimum(nv_ref[b], max_views)
    jmax = (nv + bv - 1) // bv - 1

    @pl.when(j == 0)
    def _init():
        o_ref[...] = jnp.full_like(o_ref, -jnp.inf)

    @pl.when(j <= jmax)
    def _update():
        jb = jnp.minimum(j, jmax)
        row = jb * bv + lax.broadcasted_iota(jnp.int32, (bv, 1), 0)
        blk = jnp.where(row < nv, x_ref[0], -jnp.inf)
        part = blk[0:8]
        for r in range(1, bv // 8):
            part = jnp.maximum(part, blk[r * 8:(r + 1) * 8])
        o_ref[0] = jnp.maximum(o_ref[0], jnp.max(part, axis=0, keepdims=True))


def _linear_body(k_ref, w_ref, bias_ref, o_ref):
    out = lax.dot_general(
        k_ref[...], w_ref[...],
        dimension_numbers=(((1,), (1,)), ((), ())),
        preferred_element_type=jnp.float32,
    )
    o_ref[...] = out + bias_ref[...]


def kernel(batch_size, max_num_views, num_views, x, W, b):
    B, V, D = x.shape
    O = W.shape[0]
    b_tc = B - B_SC
    fb = D // 32
    nv32 = num_views.astype(jnp.int32)

    sc_pool = functools.partial(
        pl.kernel,
        mesh=plsc.VectorSubcoreMesh(core_axis_name="c", subcore_axis_name="s"),
        out_type=jax.ShapeDtypeStruct((B_SC, D), jnp.float32),
        scratch_types=[
            pltpu.VMEM((32,), jnp.int32),
            pltpu.VMEM((2, CV, fb), jnp.float32),
            pltpu.VMEM((fb,), jnp.float32),
            pltpu.SemaphoreType.DMA((2,)),
        ],
    )(functools.partial(_sc_pool_body, cv=CV, fb=fb, b0=b_tc, nb=B_SC, V=V))
    k_sc = sc_pool(nv32, x)

    def x_index(bi, j, nv_ref):
        nv = jnp.minimum(nv_ref[bi], V)
        jmax = (nv + BV - 1) // BV - 1
        return bi, jnp.minimum(j, jmax), 0

    pool = pl.pallas_call(
        functools.partial(_pool_body, bv=BV, max_views=V),
        grid_spec=pltpu.PrefetchScalarGridSpec(
            num_scalar_prefetch=1,
            grid=(b_tc, V // BV),
            in_specs=[pl.BlockSpec((1, BV, D), x_index)],
            out_specs=pl.BlockSpec((1, 1, D), lambda bi, j, nv_ref: (bi, 0, 0)),
        ),
        out_shape=jax.ShapeDtypeStruct((b_tc, 1, D), jnp.float32),
        compiler_params=pltpu.CompilerParams(
            dimension_semantics=("arbitrary", "arbitrary"),
        ),
    )
    k_tc = pool(nv32, x).reshape(b_tc, D)
    k = jnp.concatenate([k_tc, k_sc], axis=0)

    bias = b.reshape(1, O)
    linear = pl.pallas_call(
        _linear_body,
        grid=(O // BO,),
        in_specs=[
            pl.BlockSpec((B, D), lambda o: (0, 0)),
            pl.BlockSpec((BO, D), lambda o: (o, 0)),
            pl.BlockSpec((1, BO), lambda o: (0, o)),
        ],
        out_specs=pl.BlockSpec((B, BO), lambda o: (0, o)),
        out_shape=jax.ShapeDtypeStruct((B, O), jnp.float32),
        compiler_params=pltpu.CompilerParams(
            dimension_semantics=("arbitrary",),
        ),
    )
    logits = linear(k, W, bias)
    return (logits, k)


# speedup vs baseline: 1.0691x; 1.0691x over previous
import functools

import jax
import jax.numpy as jnp
from jax import lax
from jax.experimental import pallas as pl
from jax.experimental.pallas import tpu as pltpu

BV = 256
BO = 1024


def _pool_body(nv_ref, x_ref, o_ref, *, bv, max_views):
    b = pl.program_id(0)
    j = pl.program_id(1)
    nv = jnp.minimum(nv_ref[b], max_views)
    jmax = (nv + bv - 1) // bv - 1

    @pl.when(j == 0)
    def _init():
        o_ref[...] = jnp.full_like(o_ref, -jnp.inf)

    @pl.when(j <= jmax)
    def _update():
        jb = jnp.minimum(j, jmax)
        row = jb * bv + lax.broadcasted_iota(jnp.int32, (bv, 1), 0)
        blk = jnp.where(row < nv, x_ref[0], -jnp.inf)
        part = blk[0:8]
        for r in range(1, bv // 8):
            part = jnp.maximum(part, blk[r * 8:(r + 1) * 8])
        o_ref[0] = jnp.maximum(o_ref[0], jnp.max(part, axis=0, keepdims=True))


def _linear_body(k_ref, w_ref, bias_ref, o_ref):
    out = lax.dot_general(
        k_ref[...], w_ref[...],
        dimension_numbers=(((1,), (1,)), ((), ())),
        preferred_element_type=jnp.float32,
    )
    o_ref[...] = out + bias_ref[...]


def kernel(batch_size, max_num_views, num_views, x, W, b):
    B, V, D = x.shape
    O = W.shape[0]

    def x_index(bi, j, nv_ref):
        nv = jnp.minimum(nv_ref[bi], V)
        jmax = (nv + BV - 1) // BV - 1
        return bi, jnp.minimum(j, jmax), 0

    pool = pl.pallas_call(
        functools.partial(_pool_body, bv=BV, max_views=V),
        grid_spec=pltpu.PrefetchScalarGridSpec(
            num_scalar_prefetch=1,
            grid=(B, V // BV),
            in_specs=[pl.BlockSpec((1, BV, D), x_index)],
            out_specs=pl.BlockSpec((1, 1, D), lambda bi, j, nv_ref: (bi, 0, 0)),
        ),
        out_shape=jax.ShapeDtypeStruct((B, 1, D), jnp.float32),
        compiler_params=pltpu.CompilerParams(
            dimension_semantics=("arbitrary", "arbitrary"),
        ),
    )
    k = pool(num_views.astype(jnp.int32), x).reshape(B, D)

    bias = b.reshape(1, O)
    linear = pl.pallas_call(
        _linear_body,
        grid=(O // BO,),
        in_specs=[
            pl.BlockSpec((B, D), lambda o: (0, 0)),
            pl.BlockSpec((BO, D), lambda o: (o, 0)),
            pl.BlockSpec((1, BO), lambda o: (0, o)),
        ],
        out_specs=pl.BlockSpec((B, BO), lambda o: (0, o)),
        out_shape=jax.ShapeDtypeStruct((B, O), jnp.float32),
        compiler_params=pltpu.CompilerParams(
            dimension_semantics=("arbitrary",),
        ),
    )
    logits = linear(k, W, bias)
    return (logits, k)


# speedup vs baseline: 1.1624x; 1.0872x over previous
import functools

import jax
import jax.numpy as jnp
from jax import lax
from jax.experimental import pallas as pl
from jax.experimental.pallas import tpu as pltpu

BV = 256
BO = 512


def _pool_body(nv_ref, x_ref, o_ref, *, bv, max_views, spb):
    bi = pl.program_id(0)
    j = pl.program_id(1)

    @pl.when(j == 0)
    def _init():
        o_ref[...] = jnp.full_like(o_ref, -jnp.inf)

    for l in range(spb):
        nv = jnp.minimum(nv_ref[bi * spb + l], max_views)
        jmax = (nv + bv - 1) // bv - 1

        @pl.when(j <= jmax)
        def _update(l=l, nv=nv):
            row = j * bv + lax.broadcasted_iota(jnp.int32, (bv, 1), 0)
            blk = jnp.where(row < nv, x_ref[l], -jnp.inf)
            part = blk[0:8]
            for r in range(1, bv // 8):
                part = jnp.maximum(part, blk[r * 8:(r + 1) * 8])
            o_ref[l] = jnp.maximum(o_ref[l],
                                   jnp.max(part, axis=0, keepdims=True))


def _linear_body(k_ref, w_ref, bias_ref, o_ref):
    out = lax.dot_general(
        k_ref[...], w_ref[...],
        dimension_numbers=(((1,), (1,)), ((), ())),
        preferred_element_type=jnp.float32,
    )
    o_ref[...] = out + bias_ref[...]


def kernel(batch_size, max_num_views, num_views, x, W, b):
    B, V, D = x.shape
    O = W.shape[0]

    SPB = 2

    pool = pl.pallas_call(
        functools.partial(_pool_body, bv=BV, max_views=V, spb=SPB),
        grid_spec=pltpu.PrefetchScalarGridSpec(
            num_scalar_prefetch=1,
            grid=(B // SPB, V // BV),
            in_specs=[pl.BlockSpec((SPB, BV, D),
                                   lambda bi, j, nv_ref: (bi, j, 0))],
            out_specs=pl.BlockSpec((SPB, 1, D),
                                   lambda bi, j, nv_ref: (bi, 0, 0)),
        ),
        out_shape=jax.ShapeDtypeStruct((B, 1, D), jnp.float32),
        compiler_params=pltpu.CompilerParams(
            dimension_semantics=("arbitrary", "arbitrary"),
        ),
    )
    k = pool(num_views.astype(jnp.int32), x).reshape(B, D)

    bias = b.reshape(1, O)
    linear = pl.pallas_call(
        _linear_body,
        grid=(O // BO,),
        in_specs=[
            pl.BlockSpec((B, D), lambda o: (0, 0)),
            pl.BlockSpec((BO, D), lambda o: (o, 0)),
            pl.BlockSpec((1, BO), lambda o: (0, o)),
        ],
        out_specs=pl.BlockSpec((B, BO), lambda o: (0, o)),
        out_shape=jax.ShapeDtypeStruct((B, O), jnp.float32),
        compiler_params=pltpu.CompilerParams(
            dimension_semantics=("arbitrary",),
        ),
    )
    logits = linear(k, W, bias)
    return (logits, k)
